# baseline (speedup 1.0000x reference)
import functools

import jax
import jax.numpy as jnp
from jax import lax
from jax.experimental import pallas as pl
from jax.experimental.pallas import tpu as pltpu
from jax.experimental.pallas import tpu_sc as plsc

DIM = 64
BATCH = 16384
HIST = 50
SUB = 256
NSUB_H = 2
N_C = HIST * NSUB_H


@functools.lru_cache(maxsize=None)
def _build():
    info = plsc.get_sparse_core_info()
    nc = info.num_cores
    per_w = BATCH // (nc * info.num_subcores)
    assert per_w == NSUB_H * SUB

    mesh = plsc.VectorSubcoreMesh(core_axis_name="c", subcore_axis_name="s")

    @functools.partial(
        pl.kernel,
        mesh=mesh,
        out_type=jax.ShapeDtypeStruct((HIST, DIM, BATCH), jnp.float32),
        scratch_types=[
            pltpu.VMEM((2, per_w), jnp.int32),
            pltpu.VMEM((2, 2 * NSUB_H, 128), jnp.int32),
            pltpu.VMEM((2, per_w), jnp.int32),
            pltpu.VMEM((SUB, 128), jnp.float32),
            pltpu.VMEM((SUB, 128), jnp.float32),
            pltpu.VMEM((DIM, 128), jnp.float32),
            pltpu.VMEM((DIM, 128), jnp.float32),
            pltpu.SemaphoreType.DMA,
            pltpu.SemaphoreType.DMA,
            pltpu.SemaphoreType.DMA,
            pltpu.SemaphoreType.DMA,
            pltpu.SemaphoreType.DMA,
        ],
        compiler_params=pltpu.CompilerParams(
            use_tc_tiling_on_sc=False, needs_layout_passes=False),
    )
    def emb(tokt, table2, out3, tidx, widx, colb, rows0, rows1, tr0, tr1,
            sg0, sg1, so0, so1, si):
        wid = lax.axis_index("s") * nc + lax.axis_index("c")
        b0 = wid * per_w
        rows = (rows0, rows1)
        tr = (tr0, tr1)
        sg = (sg0, sg1)
        so = (so0, so1)
        iota16 = lax.iota(jnp.int32, 16)

        def idx_start(h, slot):
            pltpu.async_copy(tokt.at[h, pl.ds(b0, per_w)], tidx.at[slot], si)

        def prep(h, slot):
            pltpu.make_async_copy(tokt.at[0, pl.ds(0, per_w)],
                                  tidx.at[slot], si).wait()
            for j in range(2 * NSUB_H):
                for k in range(8):
                    t = tidx[slot, pl.ds(j * 128 + k * 16, 16)]
                    widx[slot, j, pl.ds(k * 16, 16)] = (
                        lax.shift_right_logical(t, 1))
                    colb[slot, pl.ds(j * 128 + k * 16, 16)] = (
                        lax.shift_left(jnp.bitwise_and(t, 1), 6))

        def gather_start(c, b):
            slot = jnp.bitwise_and(c // NSUB_H, 1)
            j = jnp.remainder(c, NSUB_H) * 2
            pltpu.async_copy(table2.at[widx.at[slot, j]],
                             rows[b].at[pl.ds(0, 128)], sg[b])
            pltpu.async_copy(table2.at[widx.at[slot, j + 1]],
                             rows[b].at[pl.ds(128, 128)], sg[b])

        def gather_wait(b):
            pltpu.make_async_copy(table2.at[pl.ds(0, SUB)], rows[b],
                                  sg[b]).wait()

        def transpose(c, b, half):
            slot = jnp.bitwise_and(c // NSUB_H, 1)
            q = jnp.remainder(c, NSUB_H) * SUB + half * 128
            for kb in range(8):
                cb = colb[slot, pl.ds(q + kb * 16, 16)]
                rowv = half * 128 + kb * 16 + iota16
                for dg in range(0, DIM, 16):
                    vs = [plsc.load_gather(rows[b], [rowv, cb + (dg + i)])
                          for i in range(16)]
                    for i in range(16):
                        tr[half].at[dg + i][pl.ds(kb * 16, 16)] = vs[i]

        def out_start(c, half):
            h = c // NSUB_H
            bb = b0 + jnp.remainder(c, NSUB_H) * SUB + half * 128
            pltpu.async_copy(tr[half], out3.at[h, :, pl.ds(bb, 128)],
                             so[half])

        def out_drain(half):
            pltpu.make_async_copy(tr[half], out3.at[0, :, pl.ds(0, 128)],
                                  so[half]).wait()

        idx_start(0, 0)
        prep(0, 0)
        idx_start(1, 1)
        gather_start(0, 0)

        def body(step, carry):
            for u in (0, 1):
                c = 2 * step + u
                nb = 1 - u

                @pl.when(c + 1 < N_C)
                def _ahead():
                    @pl.when(jnp.remainder(c + 1, NSUB_H) == 0)
                    def _prep():
                        nh = (c + 1) // NSUB_H
                        prep(nh, jnp.bitwise_and(nh, 1))

                        @pl.when(nh + 1 < HIST)
                        def _istart():
                            idx_start(nh + 1, jnp.bitwise_and(nh + 1, 1))

                    gather_start(c + 1, nb)

                gather_wait(u)
                for half in (0, 1):
                    @pl.when(c >= 1)
                    def _reuse(half=half):
                        out_drain(half)

                    transpose(c, u, half)
                    out_start(c, half)
            return carry

        lax.fori_loop(0, N_C // 2, body, 0)
        out_drain(0)
        out_drain(1)

    return emb


@jax.jit
def kernel(token_ids, weight):
    tokt = token_ids.T.astype(jnp.int32)
    table2 = weight.reshape(weight.shape[0] // 2, 2 * weight.shape[1])
    o3 = _build()(tokt, table2)
    return o3.transpose(2, 0, 1)


# speedup vs baseline: 1.1649x; 1.1649x over previous
import functools

import jax
import jax.numpy as jnp
from jax import lax
from jax.experimental import pallas as pl
from jax.experimental.pallas import tpu as pltpu
from jax.experimental.pallas import tpu_sc as plsc

DIM = 64
BATCH = 16384
HIST = 50
SUB = 256
NSUB_H = 2
N_C = HIST * NSUB_H


@functools.lru_cache(maxsize=None)
def _build():
    info = plsc.get_sparse_core_info()
    nc = info.num_cores
    per_w = BATCH // (nc * info.num_subcores)
    assert per_w == NSUB_H * SUB

    mesh = plsc.VectorSubcoreMesh(core_axis_name="c", subcore_axis_name="s")

    @functools.partial(
        pl.kernel,
        mesh=mesh,
        out_type=jax.ShapeDtypeStruct((HIST, DIM, BATCH), jnp.float32),
        scratch_types=[
            pltpu.VMEM((2, per_w), jnp.int32),
            pltpu.VMEM((2, 2 * NSUB_H, 128), jnp.int32),
            pltpu.VMEM((2, per_w), jnp.int32),
            pltpu.VMEM((SUB, 128), jnp.float32),
            pltpu.VMEM((SUB, 128), jnp.float32),
            pltpu.VMEM((SUB, 128), jnp.float32),
            pltpu.VMEM((DIM, 128), jnp.float32),
            pltpu.VMEM((DIM, 128), jnp.float32),
            pltpu.SemaphoreType.DMA,
            pltpu.SemaphoreType.DMA,
            pltpu.SemaphoreType.DMA,
            pltpu.SemaphoreType.DMA,
            pltpu.SemaphoreType.DMA,
            pltpu.SemaphoreType.DMA,
        ],
        compiler_params=pltpu.CompilerParams(
            use_tc_tiling_on_sc=True, needs_layout_passes=False),
    )
    def emb(tokt, table2, out3, tidx, widx, colb, rows0, rows1, rows2,
            tr0, tr1, sg0, sg1, sg2, so0, so1, si):
        wid = lax.axis_index("s") * nc + lax.axis_index("c")
        b0 = wid * per_w
        rows = (rows0, rows1, rows2)
        tr = (tr0, tr1)
        sg = (sg0, sg1, sg2)
        so = (so0, so1)
        iota16 = lax.iota(jnp.int32, 16)

        def idx_start(h, slot):
            pltpu.async_copy(tokt.at[h, pl.ds(b0, per_w)], tidx.at[slot], si)

        def prep(h, slot):
            pltpu.make_async_copy(tokt.at[0, pl.ds(0, per_w)],
                                  tidx.at[slot], si).wait()
            for j in range(2 * NSUB_H):
                for k in range(8):
                    t = tidx[slot, pl.ds(j * 128 + k * 16, 16)]
                    widx[slot, j, pl.ds(k * 16, 16)] = (
                        lax.shift_right_logical(t, 1))
                    colb[slot, pl.ds(j * 128 + k * 16, 16)] = (
                        lax.shift_left(jnp.bitwise_and(t, 1), 6))

        def gather_start(c, b):
            slot = jnp.bitwise_and(c // NSUB_H, 1)
            j = jnp.remainder(c, NSUB_H) * 2
            pltpu.async_copy(table2.at[widx.at[slot, j]],
                             rows[b].at[pl.ds(0, 128)], sg[b])
            pltpu.async_copy(table2.at[widx.at[slot, j + 1]],
                             rows[b].at[pl.ds(128, 128)], sg[b])

        def gather_wait(b):
            pltpu.make_async_copy(table2.at[pl.ds(0, SUB)], rows[b],
                                  sg[b]).wait()

        def transpose(c, b, half):
            slot = jnp.bitwise_and(c // NSUB_H, 1)
            q = jnp.remainder(c, NSUB_H) * SUB + half * 128
            for kb in range(8):
                cb = colb[slot, pl.ds(q + kb * 16, 16)]
                rowv = half * 128 + kb * 16 + iota16

                def dgbody(g, carry, cb=cb, rowv=rowv, kb=kb):
                    dg = g * 16
                    vs = [plsc.load_gather(rows[b], [rowv, cb + (dg + i)])
                          for i in range(16)]
                    for i in range(16):
                        tr[half].at[dg + i][pl.ds(kb * 16, 16)] = vs[i]
                    return carry

                lax.fori_loop(0, DIM // 16, dgbody, 0)

        def out_start(c, half):
            h = c // NSUB_H
            bb = b0 + jnp.remainder(c, NSUB_H) * SUB + half * 128
            pltpu.async_copy(tr[half], out3.at[h, :, pl.ds(bb, 128)],
                             so[half])

        def out_drain(half):
            pltpu.make_async_copy(tr[half], out3.at[0, :, pl.ds(0, 128)],
                                  so[half]).wait()

        idx_start(0, 0)
        prep(0, 0)
        idx_start(1, 1)
        prep(1, 1)
        idx_start(2, 0)
        gather_start(0, 0)
        gather_start(1, 1)

        def stage(c, u):
            @pl.when(c + 2 < N_C)
            def _ahead():
                @pl.when(jnp.remainder(c + 2, NSUB_H) == 0)
                def _prep():
                    nh = (c + 2) // NSUB_H
                    prep(nh, jnp.bitwise_and(nh, 1))

                    @pl.when(nh + 1 < HIST)
                    def _istart():
                        idx_start(nh + 1, jnp.bitwise_and(nh + 1, 1))

                gather_start(c + 2, (u + 2) % 3)

            gather_wait(u)
            for half in (0, 1):
                @pl.when(c >= 1)
                def _reuse(half=half):
                    out_drain(half)

                transpose(c, u, half)
                out_start(c, half)

        def body(step, carry):
            for u in (0, 1, 2):
                stage(3 * step + u, u)
            return carry

        lax.fori_loop(0, (N_C - 1) // 3, body, 0)
        stage(N_C - 1, (N_C - 1) % 3)
        out_drain(0)
        out_drain(1)

    return emb


@jax.jit
def kernel(token_ids, weight):
    tokt = token_ids.T.astype(jnp.int32)
    table2 = weight.reshape(weight.shape[0] // 2, 2 * weight.shape[1])
    o3 = _build()(tokt, table2)
    return o3.transpose(2, 0, 1)


# speedup vs baseline: 1.2472x; 1.0706x over previous
import functools

import jax
import jax.numpy as jnp
from jax import lax
from jax.experimental import pallas as pl
from jax.experimental.pallas import tpu as pltpu
from jax.experimental.pallas import tpu_sc as plsc

DIM = 64
IDXW = 128
CHUNK = 640
J = CHUNK // IDXW


@functools.lru_cache(maxsize=None)
def _build(B: int):
    info = plsc.get_sparse_core_info()
    nc, ns = info.num_cores, info.num_subcores
    nw = nc * ns
    assert B % (nw * 2 * CHUNK) == 0
    b_per_w = B // nw
    n_chunks = b_per_w // CHUNK
    rows_per_w = b_per_w // IDXW

    mesh = plsc.VectorSubcoreMesh(core_axis_name="c", subcore_axis_name="s")

    @functools.partial(
        pl.kernel,
        mesh=mesh,
        out_type=jax.ShapeDtypeStruct((B, DIM), jnp.float32),
        scratch_types=[
            pltpu.VMEM((rows_per_w, IDXW), jnp.int32),
            pltpu.VMEM((CHUNK, DIM), jnp.float32),
            pltpu.VMEM((CHUNK, DIM), jnp.float32),
            pltpu.SemaphoreType.DMA,
            pltpu.SemaphoreType.DMA,
            pltpu.SemaphoreType.DMA,
            pltpu.SemaphoreType.DMA,
        ],
        compiler_params=pltpu.CompilerParams(use_tc_tiling_on_sc=False),
    )
    def emb(idx_hbm, table_hbm, out_hbm, idx_all, rows0, rows1,
            sg0, sg1, so0, so1):
        wid = lax.axis_index("s") * nc + lax.axis_index("c")
        row_base = wid * rows_per_w
        out_base = wid * b_per_w
        rows = (rows0, rows1)
        sg = (sg0, sg1)
        so = (so0, so1)

        pltpu.sync_copy(idx_hbm.at[pl.ds(row_base, rows_per_w)], idx_all)

        def gather_start(c, b):
            for j in range(J):
                pltpu.async_copy(
                    table_hbm.at[idx_all.at[c * J + j]],
                    rows[b].at[pl.ds(j * IDXW, IDXW)],
                    sg[b],
                )

        def gather_wait(b):
            pltpu.make_async_copy(
                table_hbm.at[pl.ds(0, CHUNK)], rows[b], sg[b]
            ).wait()

        def out_start(c, b):
            pltpu.async_copy(
                rows[b], out_hbm.at[pl.ds(out_base + c * CHUNK, CHUNK)], so[b]
            )

        def out_drain(b):
            pltpu.make_async_copy(
                rows[b], out_hbm.at[pl.ds(out_base, CHUNK)], so[b]
            ).wait()

        gather_start(0, 0)

        def body(s, carry):
            for b in (0, 1):
                c = 2 * s + b
                nb = 1 - b

                @pl.when(c + 1 < n_chunks)
                def _prefetch():
                    @pl.when(c >= 1)
                    def _reuse():
                        out_drain(nb)

                    gather_start(c + 1, nb)

                gather_wait(b)
                out_start(c, b)
            return carry

        lax.fori_loop(0, n_chunks // 2, body, 0)
        out_drain(0)
        out_drain(1)

    return emb


@jax.jit
def kernel(token_ids, weight):
    B = token_ids.size
    idx = token_ids.reshape(B // IDXW, IDXW).astype(jnp.int32)
    out = _build(B)(idx, weight)
    return out.reshape(token_ids.shape + (DIM,))
